# baseline (speedup 1.0000x reference)
import functools

import jax
import jax.numpy as jnp
from jax import lax
from jax.experimental import pallas as pl
from jax.experimental.pallas import tpu as pltpu
from jax.experimental.pallas import tpu_sc as plsc

N_NODES = 10000
N_EDGES = 320000
D = 128

NUM_CORES = 2
NUM_SUBCORES = 16
NW = NUM_CORES * NUM_SUBCORES
E_PER_W = N_EDGES // NW
CHUNK = 80
NCHUNK = 125

ACC_ROWS = N_NODES
R_PER_T = 624
R_TAIL = N_NODES - R_PER_T * NUM_SUBCORES
TAIL_OFF = R_PER_T * NUM_SUBCORES


def _sc_aggregate(x, srcs, dsts, zeros):
    mesh = plsc.VectorSubcoreMesh(core_axis_name="c", subcore_axis_name="s")

    @functools.partial(
        pl.kernel,
        mesh=mesh,
        out_type=jax.ShapeDtypeStruct((NUM_CORES, N_NODES, D), jnp.float32),
        scratch_types=[
            pltpu.VMEM((E_PER_W,), jnp.int32),
            pltpu.VMEM((NCHUNK, CHUNK), jnp.int32),
            pltpu.VMEM((CHUNK, D), jnp.float32),
            pltpu.VMEM((CHUNK, D), jnp.float32),
            pltpu.VMEM_SHARED((ACC_ROWS, D), jnp.float32),
            pltpu.SemaphoreType.DMA,
            pltpu.SemaphoreType.DMA,
            pltpu.SemaphoreType.DMA,
            pltpu.SemaphoreType.DMA,
        ],
    )
    def agg_kernel(x_hbm, src_hbm, dst_hbm, zeros_hbm, out_hbm,
                   src_v, dst_v, rows0, rows1, agg_sh, sem0, sem1,
                   ssem0, ssem1):
        c = lax.axis_index("c")
        s = lax.axis_index("s")
        wid = c * NUM_SUBCORES + s

        pltpu.sync_copy(src_hbm.at[pl.ds(wid * E_PER_W, E_PER_W)], src_v)
        pltpu.sync_copy(dst_hbm.at[wid], dst_v)
        pltpu.async_copy(x_hbm.at[src_v.at[pl.ds(0, CHUNK)]], rows0, sem0)
        pltpu.async_copy(x_hbm.at[src_v.at[pl.ds(CHUNK, CHUNK)]], rows1, sem1)

        @pl.when(c == 0)
        def _():
            pltpu.sync_copy(x_hbm.at[pl.ds(s * R_PER_T, R_PER_T)],
                            agg_sh.at[pl.ds(s * R_PER_T, R_PER_T)])

            @pl.when(s == NUM_SUBCORES - 1)
            def _():
                pltpu.sync_copy(x_hbm.at[pl.ds(TAIL_OFF, R_TAIL)],
                                agg_sh.at[pl.ds(TAIL_OFF, R_TAIL)])

        @pl.when(c != 0)
        def _():
            pltpu.sync_copy(zeros_hbm, agg_sh.at[pl.ds(s * R_PER_T, R_PER_T)])

            @pl.when(s == NUM_SUBCORES - 1)
            def _():
                pltpu.sync_copy(zeros_hbm.at[pl.ds(0, R_TAIL)],
                                agg_sh.at[pl.ds(TAIL_OFF, R_TAIL)])

        plsc.subcore_barrier()

        def src_at(i):
            return src_v.at[pl.ds(i * CHUNK, CHUNK)]

        def pair_body(j, carry):
            i = 2 * j
            pltpu.make_async_copy(x_hbm.at[src_at(i)], rows0, sem0).wait()
            sc0 = pltpu.async_copy(rows0, agg_sh.at[dst_v.at[i]], ssem0,
                                   add=True)
            pltpu.make_async_copy(x_hbm.at[src_at(i + 1)], rows1, sem1).wait()
            sc1 = pltpu.async_copy(rows1, agg_sh.at[dst_v.at[i + 1]], ssem1,
                                   add=True)
            sc0.wait()
            pltpu.async_copy(x_hbm.at[src_at(i + 2)], rows0, sem0)
            sc1.wait()
            pltpu.async_copy(x_hbm.at[src_at(i + 3)], rows1, sem1)
            return carry

        lax.fori_loop(0, (NCHUNK - 3) // 2, pair_body, 0)

        i = NCHUNK - 3
        pltpu.make_async_copy(x_hbm.at[src_at(i)], rows0, sem0).wait()
        sc0 = pltpu.async_copy(rows0, agg_sh.at[dst_v.at[i]], ssem0, add=True)
        pltpu.make_async_copy(x_hbm.at[src_at(i + 1)], rows1, sem1).wait()
        sc1 = pltpu.async_copy(rows1, agg_sh.at[dst_v.at[i + 1]], ssem1,
                               add=True)
        sc0.wait()
        pltpu.async_copy(x_hbm.at[src_at(i + 2)], rows0, sem0)
        sc1.wait()
        pltpu.make_async_copy(x_hbm.at[src_at(i + 2)], rows0, sem0).wait()
        pltpu.sync_copy(rows0, agg_sh.at[dst_v.at[i + 2]], add=True)

        plsc.subcore_barrier()

        pltpu.sync_copy(agg_sh.at[pl.ds(s * R_PER_T, R_PER_T)],
                        out_hbm.at[c, pl.ds(s * R_PER_T, R_PER_T)])

        @pl.when(s == NUM_SUBCORES - 1)
        def _():
            pltpu.sync_copy(agg_sh.at[pl.ds(TAIL_OFF, R_TAIL)],
                            out_hbm.at[c, pl.ds(TAIL_OFF, R_TAIL)])

    return agg_kernel(x, srcs, dsts, zeros)


def _tc_mlp(parts, W1, b1, W2, b2):
    BR = 1000
    grid = N_NODES // BR

    def mlp_body(p_ref, w1_ref, b1_ref, w2_ref, b2_ref, o_ref):
        h = p_ref[0] + p_ref[1]
        h = jnp.dot(h, w1_ref[...], preferred_element_type=jnp.float32)
        h = jnp.maximum(h + b1_ref[...], 0.0)
        h = jnp.dot(h, w2_ref[...], preferred_element_type=jnp.float32)
        o_ref[...] = h + b2_ref[...]

    return pl.pallas_call(
        mlp_body,
        grid=(grid,),
        in_specs=[
            pl.BlockSpec((NUM_CORES, BR, D), lambda i: (0, i, 0)),
            pl.BlockSpec((D, D), lambda i: (0, 0)),
            pl.BlockSpec((1, D), lambda i: (0, 0)),
            pl.BlockSpec((D, D), lambda i: (0, 0)),
            pl.BlockSpec((1, D), lambda i: (0, 0)),
        ],
        out_specs=pl.BlockSpec((BR, D), lambda i: (i, 0)),
        out_shape=jax.ShapeDtypeStruct((N_NODES, D), jnp.float32),
    )(parts, W1, b1, W2, b2)


def kernel(x, edge_index, W1, b1, W2, b2):
    ei = edge_index.astype(jnp.int32)
    srcs = ei[0]
    dsts = ei[1].reshape(NW, NCHUNK, CHUNK)
    zeros = jnp.zeros((R_PER_T, D), jnp.float32)
    parts = _sc_aggregate(x, srcs, dsts, zeros)
    return _tc_mlp(parts, W1.astype(jnp.float32), b1.reshape(1, D),
                   W2.astype(jnp.float32), b2.reshape(1, D))


# speedup vs baseline: 1.2179x; 1.2179x over previous
import functools

import jax
import jax.numpy as jnp
from jax import lax
from jax.experimental import pallas as pl
from jax.experimental.pallas import tpu as pltpu
from jax.experimental.pallas import tpu_sc as plsc

N_NODES = 10000
N_EDGES = 320000
D = 128

NUM_CORES = 2
NUM_SUBCORES = 16
NW = NUM_CORES * NUM_SUBCORES
E_PER_W = N_EDGES // NW
CHUNK = 80
NCHUNK = 125

ACC_ROWS = N_NODES
R_PER_T = 624
R_TAIL = N_NODES - R_PER_T * NUM_SUBCORES
TAIL_OFF = R_PER_T * NUM_SUBCORES


def _sc_aggregate(x, srcs, dsts, zeros):
    mesh = plsc.VectorSubcoreMesh(core_axis_name="c", subcore_axis_name="s")

    @functools.partial(
        pl.kernel,
        mesh=mesh,
        out_type=jax.ShapeDtypeStruct((NUM_CORES, N_NODES, D), jnp.float32),
        scratch_types=[
            pltpu.VMEM((E_PER_W,), jnp.int32),
            pltpu.VMEM((NCHUNK, CHUNK), jnp.int32),
            pltpu.VMEM((CHUNK, D), jnp.float32),
            pltpu.VMEM((CHUNK, D), jnp.float32),
            pltpu.VMEM_SHARED((ACC_ROWS, D), jnp.float32),
            pltpu.SemaphoreType.DMA,
            pltpu.SemaphoreType.DMA,
        ],
    )
    def agg_kernel(x_hbm, src_hbm, dst_hbm, zeros_hbm, out_hbm,
                   src_v, dst_v, rows0, rows1, agg_sh, sem0, sem1):
        c = lax.axis_index("c")
        s = lax.axis_index("s")
        wid = c * NUM_SUBCORES + s

        pltpu.sync_copy(src_hbm.at[pl.ds(wid * E_PER_W, E_PER_W)], src_v)
        pltpu.sync_copy(dst_hbm.at[wid], dst_v)
        pltpu.async_copy(x_hbm.at[src_v.at[pl.ds(0, CHUNK)]], rows0, sem0)

        @pl.when(c == 0)
        def _():
            pltpu.sync_copy(x_hbm.at[pl.ds(s * R_PER_T, R_PER_T)],
                            agg_sh.at[pl.ds(s * R_PER_T, R_PER_T)])

            @pl.when(s == NUM_SUBCORES - 1)
            def _():
                pltpu.sync_copy(x_hbm.at[pl.ds(TAIL_OFF, R_TAIL)],
                                agg_sh.at[pl.ds(TAIL_OFF, R_TAIL)])

        @pl.when(c != 0)
        def _():
            pltpu.sync_copy(zeros_hbm, agg_sh.at[pl.ds(s * R_PER_T, R_PER_T)])

            @pl.when(s == NUM_SUBCORES - 1)
            def _():
                pltpu.sync_copy(zeros_hbm.at[pl.ds(0, R_TAIL)],
                                agg_sh.at[pl.ds(TAIL_OFF, R_TAIL)])

        plsc.subcore_barrier()

        def src_at(i):
            return src_v.at[pl.ds(i * CHUNK, CHUNK)]

        def pair_body(j, carry):
            i = 2 * j
            pltpu.async_copy(x_hbm.at[src_at(i + 1)], rows1, sem1)
            pltpu.make_async_copy(x_hbm.at[src_at(i)], rows0, sem0).wait()
            pltpu.sync_copy(rows0, agg_sh.at[dst_v.at[i]], add=True)
            pltpu.async_copy(x_hbm.at[src_at(i + 2)], rows0, sem0)
            pltpu.make_async_copy(x_hbm.at[src_at(i + 1)], rows1, sem1).wait()
            pltpu.sync_copy(rows1, agg_sh.at[dst_v.at[i + 1]], add=True)
            return carry

        lax.fori_loop(0, (NCHUNK - 1) // 2, pair_body, 0)

        last = NCHUNK - 1
        pltpu.make_async_copy(x_hbm.at[src_at(last)], rows0, sem0).wait()
        pltpu.sync_copy(rows0, agg_sh.at[dst_v.at[last]], add=True)

        plsc.subcore_barrier()

        pltpu.sync_copy(agg_sh.at[pl.ds(s * R_PER_T, R_PER_T)],
                        out_hbm.at[c, pl.ds(s * R_PER_T, R_PER_T)])

        @pl.when(s == NUM_SUBCORES - 1)
        def _():
            pltpu.sync_copy(agg_sh.at[pl.ds(TAIL_OFF, R_TAIL)],
                            out_hbm.at[c, pl.ds(TAIL_OFF, R_TAIL)])

    return agg_kernel(x, srcs, dsts, zeros)


def _tc_mlp(parts, W1, b1, W2, b2):
    BR = 1000
    grid = N_NODES // BR

    def mlp_body(p_ref, w1_ref, b1_ref, w2_ref, b2_ref, o_ref):
        h = p_ref[0] + p_ref[1]
        h = jnp.dot(h, w1_ref[...], preferred_element_type=jnp.float32)
        h = jnp.maximum(h + b1_ref[...], 0.0)
        h = jnp.dot(h, w2_ref[...], preferred_element_type=jnp.float32)
        o_ref[...] = h + b2_ref[...]

    return pl.pallas_call(
        mlp_body,
        grid=(grid,),
        in_specs=[
            pl.BlockSpec((NUM_CORES, BR, D), lambda i: (0, i, 0)),
            pl.BlockSpec((D, D), lambda i: (0, 0)),
            pl.BlockSpec((1, D), lambda i: (0, 0)),
            pl.BlockSpec((D, D), lambda i: (0, 0)),
            pl.BlockSpec((1, D), lambda i: (0, 0)),
        ],
        out_specs=pl.BlockSpec((BR, D), lambda i: (i, 0)),
        out_shape=jax.ShapeDtypeStruct((N_NODES, D), jnp.float32),
    )(parts, W1, b1, W2, b2)


def kernel(x, edge_index, W1, b1, W2, b2):
    ei = edge_index.astype(jnp.int32)
    srcs = ei[0]
    dsts = ei[1].reshape(NW, NCHUNK, CHUNK)
    zeros = jnp.zeros((R_PER_T, D), jnp.float32)
    parts = _sc_aggregate(x, srcs, dsts, zeros)
    return _tc_mlp(parts, W1.astype(jnp.float32), b1.reshape(1, D),
                   W2.astype(jnp.float32), b2.reshape(1, D))


# speedup vs baseline: 1.2466x; 1.0236x over previous
import functools

import jax
import jax.numpy as jnp
from jax import lax
from jax.experimental import pallas as pl
from jax.experimental.pallas import tpu as pltpu
from jax.experimental.pallas import tpu_sc as plsc

N_NODES = 10000
N_EDGES = 320000
D = 128

NUM_CORES = 2
NUM_SUBCORES = 16
NW = NUM_CORES * NUM_SUBCORES
E_PER_W = N_EDGES // NW
CHUNK = 80
NCHUNK = 125

ACC_ROWS = N_NODES
R_PER_T = 624
R_TAIL = N_NODES - R_PER_T * NUM_SUBCORES
TAIL_OFF = R_PER_T * NUM_SUBCORES


def _sc_aggregate(x, srcs, dsts, zeros):
    mesh = plsc.VectorSubcoreMesh(core_axis_name="c", subcore_axis_name="s")

    @functools.partial(
        pl.kernel,
        mesh=mesh,
        out_type=jax.ShapeDtypeStruct((NUM_CORES, N_NODES, D), jnp.float32),
        scratch_types=[
            pltpu.VMEM((E_PER_W,), jnp.int32),
            pltpu.VMEM((NCHUNK, CHUNK), jnp.int32),
            pltpu.VMEM((CHUNK, D), jnp.float32),
            pltpu.VMEM((CHUNK, D), jnp.float32),
            pltpu.VMEM_SHARED((ACC_ROWS, D), jnp.float32),
            pltpu.SemaphoreType.DMA,
            pltpu.SemaphoreType.DMA,
        ],
        compiler_params=pltpu.CompilerParams(use_tc_tiling_on_sc=False),
    )
    def agg_kernel(x_hbm, src_hbm, dst_hbm, zeros_hbm, out_hbm,
                   src_v, dst_v, rows0, rows1, agg_sh, sem0, sem1):
        c = lax.axis_index("c")
        s = lax.axis_index("s")
        wid = c * NUM_SUBCORES + s

        pltpu.sync_copy(src_hbm.at[pl.ds(wid * E_PER_W, E_PER_W)], src_v)
        pltpu.sync_copy(dst_hbm.at[wid], dst_v)
        pltpu.async_copy(x_hbm.at[src_v.at[pl.ds(0, CHUNK)]], rows0, sem0)

        @pl.when(c == 0)
        def _():
            pltpu.sync_copy(x_hbm.at[pl.ds(s * R_PER_T, R_PER_T)],
                            agg_sh.at[pl.ds(s * R_PER_T, R_PER_T)])

            @pl.when(s == NUM_SUBCORES - 1)
            def _():
                pltpu.sync_copy(x_hbm.at[pl.ds(TAIL_OFF, R_TAIL)],
                                agg_sh.at[pl.ds(TAIL_OFF, R_TAIL)])

        @pl.when(c != 0)
        def _():
            pltpu.sync_copy(zeros_hbm, agg_sh.at[pl.ds(s * R_PER_T, R_PER_T)])

            @pl.when(s == NUM_SUBCORES - 1)
            def _():
                pltpu.sync_copy(zeros_hbm.at[pl.ds(0, R_TAIL)],
                                agg_sh.at[pl.ds(TAIL_OFF, R_TAIL)])

        plsc.subcore_barrier()

        def src_at(i):
            return src_v.at[pl.ds(i * CHUNK, CHUNK)]

        def pair_body(j, carry):
            i = 2 * j
            pltpu.async_copy(x_hbm.at[src_at(i + 1)], rows1, sem1)
            pltpu.make_async_copy(x_hbm.at[src_at(i)], rows0, sem0).wait()
            pltpu.sync_copy(rows0, agg_sh.at[dst_v.at[i]], add=True)
            pltpu.async_copy(x_hbm.at[src_at(i + 2)], rows0, sem0)
            pltpu.make_async_copy(x_hbm.at[src_at(i + 1)], rows1, sem1).wait()
            pltpu.sync_copy(rows1, agg_sh.at[dst_v.at[i + 1]], add=True)
            return carry

        lax.fori_loop(0, (NCHUNK - 1) // 2, pair_body, 0)

        last = NCHUNK - 1
        pltpu.make_async_copy(x_hbm.at[src_at(last)], rows0, sem0).wait()
        pltpu.sync_copy(rows0, agg_sh.at[dst_v.at[last]], add=True)

        plsc.subcore_barrier()

        pltpu.sync_copy(agg_sh.at[pl.ds(s * R_PER_T, R_PER_T)],
                        out_hbm.at[c, pl.ds(s * R_PER_T, R_PER_T)])

        @pl.when(s == NUM_SUBCORES - 1)
        def _():
            pltpu.sync_copy(agg_sh.at[pl.ds(TAIL_OFF, R_TAIL)],
                            out_hbm.at[c, pl.ds(TAIL_OFF, R_TAIL)])

    return agg_kernel(x, srcs, dsts, zeros)


def _tc_mlp(parts, W1, b1, W2, b2):
    BR = 1000
    grid = N_NODES // BR

    def mlp_body(p_ref, w1_ref, b1_ref, w2_ref, b2_ref, o_ref):
        h = p_ref[0] + p_ref[1]
        h = jnp.dot(h, w1_ref[...], preferred_element_type=jnp.float32)
        h = jnp.maximum(h + b1_ref[...], 0.0)
        h = jnp.dot(h, w2_ref[...], preferred_element_type=jnp.float32)
        o_ref[...] = h + b2_ref[...]

    return pl.pallas_call(
        mlp_body,
        grid=(grid,),
        in_specs=[
            pl.BlockSpec((NUM_CORES, BR, D), lambda i: (0, i, 0)),
            pl.BlockSpec((D, D), lambda i: (0, 0)),
            pl.BlockSpec((1, D), lambda i: (0, 0)),
            pl.BlockSpec((D, D), lambda i: (0, 0)),
            pl.BlockSpec((1, D), lambda i: (0, 0)),
        ],
        out_specs=pl.BlockSpec((BR, D), lambda i: (i, 0)),
        out_shape=jax.ShapeDtypeStruct((N_NODES, D), jnp.float32),
    )(parts, W1, b1, W2, b2)


def kernel(x, edge_index, W1, b1, W2, b2):
    ei = edge_index.astype(jnp.int32)
    srcs = ei[0]
    dsts = ei[1].reshape(NW, NCHUNK, CHUNK)
    zeros = jnp.zeros((R_PER_T, D), jnp.float32)
    parts = _sc_aggregate(x, srcs, dsts, zeros)
    return _tc_mlp(parts, W1.astype(jnp.float32), b1.reshape(1, D),
                   W2.astype(jnp.float32), b2.reshape(1, D))


# speedup vs baseline: 1.3318x; 1.0683x over previous
import functools

import jax
import jax.numpy as jnp
from jax import lax
from jax.experimental import pallas as pl
from jax.experimental.pallas import tpu as pltpu
from jax.experimental.pallas import tpu_sc as plsc

N_NODES = 10000
N_EDGES = 320000
D = 128

NUM_CORES = 2
NUM_SUBCORES = 16
NW = NUM_CORES * NUM_SUBCORES
E_PER_W = N_EDGES // NW
CHUNK = 80
NCHUNK = 125

ACC_ROWS = N_NODES
R_PER_T = 624
R_TAIL = N_NODES - R_PER_T * NUM_SUBCORES
TAIL_OFF = R_PER_T * NUM_SUBCORES


def _sc_aggregate(x, ei, zeros):
    mesh = plsc.VectorSubcoreMesh(core_axis_name="c", subcore_axis_name="s")

    @functools.partial(
        pl.kernel,
        mesh=mesh,
        out_type=jax.ShapeDtypeStruct((NUM_CORES, N_NODES, D), jnp.float32),
        scratch_types=[
            pltpu.VMEM((E_PER_W,), jnp.int32),
            pltpu.VMEM((E_PER_W,), jnp.int32),
            pltpu.VMEM((CHUNK, D), jnp.float32),
            pltpu.VMEM((CHUNK, D), jnp.float32),
            pltpu.VMEM_SHARED((ACC_ROWS, D), jnp.float32),
            pltpu.SemaphoreType.DMA,
            pltpu.SemaphoreType.DMA,
        ],
        compiler_params=pltpu.CompilerParams(use_tc_tiling_on_sc=False),
    )
    def agg_kernel(x_hbm, ei_hbm, zeros_hbm, out_hbm,
                   src_v, dst_v, rows0, rows1, agg_sh, sem0, sem1):
        c = lax.axis_index("c")
        s = lax.axis_index("s")
        wid = c * NUM_SUBCORES + s

        pltpu.sync_copy(ei_hbm.at[0, pl.ds(wid * E_PER_W, E_PER_W)], src_v)
        pltpu.sync_copy(ei_hbm.at[1, pl.ds(wid * E_PER_W, E_PER_W)], dst_v)
        pltpu.async_copy(x_hbm.at[src_v.at[pl.ds(0, CHUNK)]], rows0, sem0)

        @pl.when(c == 0)
        def _():
            pltpu.sync_copy(x_hbm.at[pl.ds(s * R_PER_T, R_PER_T)],
                            agg_sh.at[pl.ds(s * R_PER_T, R_PER_T)])

            @pl.when(s == NUM_SUBCORES - 1)
            def _():
                pltpu.sync_copy(x_hbm.at[pl.ds(TAIL_OFF, R_TAIL)],
                                agg_sh.at[pl.ds(TAIL_OFF, R_TAIL)])

        @pl.when(c != 0)
        def _():
            pltpu.sync_copy(zeros_hbm, agg_sh.at[pl.ds(s * R_PER_T, R_PER_T)])

            @pl.when(s == NUM_SUBCORES - 1)
            def _():
                pltpu.sync_copy(zeros_hbm.at[pl.ds(0, R_TAIL)],
                                agg_sh.at[pl.ds(TAIL_OFF, R_TAIL)])

        plsc.subcore_barrier()

        def src_at(i):
            return src_v.at[pl.ds(i * CHUNK, CHUNK)]

        def dst_at(i):
            return dst_v.at[pl.ds(i * CHUNK, CHUNK)]

        def pair_body(j, carry):
            i = 2 * j
            pltpu.async_copy(x_hbm.at[src_at(i + 1)], rows1, sem1)
            pltpu.make_async_copy(x_hbm.at[src_at(i)], rows0, sem0).wait()
            pltpu.sync_copy(rows0, agg_sh.at[dst_at(i)], add=True)
            pltpu.async_copy(x_hbm.at[src_at(i + 2)], rows0, sem0)
            pltpu.make_async_copy(x_hbm.at[src_at(i + 1)], rows1, sem1).wait()
            pltpu.sync_copy(rows1, agg_sh.at[dst_at(i + 1)], add=True)
            return carry

        lax.fori_loop(0, (NCHUNK - 1) // 2, pair_body, 0)

        last = NCHUNK - 1
        pltpu.make_async_copy(x_hbm.at[src_at(last)], rows0, sem0).wait()
        pltpu.sync_copy(rows0, agg_sh.at[dst_at(last)], add=True)

        plsc.subcore_barrier()

        pltpu.sync_copy(agg_sh.at[pl.ds(s * R_PER_T, R_PER_T)],
                        out_hbm.at[c, pl.ds(s * R_PER_T, R_PER_T)])

        @pl.when(s == NUM_SUBCORES - 1)
        def _():
            pltpu.sync_copy(agg_sh.at[pl.ds(TAIL_OFF, R_TAIL)],
                            out_hbm.at[c, pl.ds(TAIL_OFF, R_TAIL)])

    return agg_kernel(x, ei, zeros)


def _tc_mlp(parts, W1, b1, W2, b2):
    BR = 1000
    grid = N_NODES // BR

    def mlp_body(p_ref, w1_ref, b1_ref, w2_ref, b2_ref, o_ref):
        h = p_ref[0] + p_ref[1]
        h = jnp.dot(h, w1_ref[...], preferred_element_type=jnp.float32)
        h = jnp.maximum(h + b1_ref[...], 0.0)
        h = jnp.dot(h, w2_ref[...], preferred_element_type=jnp.float32)
        o_ref[...] = h + b2_ref[...]

    return pl.pallas_call(
        mlp_body,
        grid=(grid,),
        in_specs=[
            pl.BlockSpec((NUM_CORES, BR, D), lambda i: (0, i, 0)),
            pl.BlockSpec((D, D), lambda i: (0, 0)),
            pl.BlockSpec((1, D), lambda i: (0, 0)),
            pl.BlockSpec((D, D), lambda i: (0, 0)),
            pl.BlockSpec((1, D), lambda i: (0, 0)),
        ],
        out_specs=pl.BlockSpec((BR, D), lambda i: (i, 0)),
        out_shape=jax.ShapeDtypeStruct((N_NODES, D), jnp.float32),
    )(parts, W1, b1, W2, b2)


def kernel(x, edge_index, W1, b1, W2, b2):
    ei = edge_index.astype(jnp.int32)
    zeros = jnp.zeros((R_PER_T, D), jnp.float32)
    parts = _sc_aggregate(x, ei, zeros)
    return _tc_mlp(parts, W1.astype(jnp.float32), b1.reshape(1, D),
                   W2.astype(jnp.float32), b2.reshape(1, D))


# speedup vs baseline: 1.3474x; 1.0118x over previous
import functools

import jax
import jax.numpy as jnp
from jax import lax
from jax.experimental import pallas as pl
from jax.experimental.pallas import tpu as pltpu
from jax.experimental.pallas import tpu_sc as plsc

N_NODES = 10000
N_EDGES = 320000
D = 128

NUM_CORES = 2
NUM_SUBCORES = 16
NW = NUM_CORES * NUM_SUBCORES
E_PER_W = N_EDGES // NW
CHUNK = 80
NCHUNK = 125

ACC_ROWS = N_NODES
R_PER_T = 624
R_TAIL = N_NODES - R_PER_T * NUM_SUBCORES
TAIL_OFF = R_PER_T * NUM_SUBCORES


def _sc_aggregate(x, ei, zeros):
    mesh = plsc.VectorSubcoreMesh(core_axis_name="c", subcore_axis_name="s")

    @functools.partial(
        pl.kernel,
        mesh=mesh,
        out_type=jax.ShapeDtypeStruct((NUM_CORES, N_NODES, D), jnp.float32),
        scratch_types=[
            pltpu.VMEM((E_PER_W,), jnp.int32),
            pltpu.VMEM((E_PER_W,), jnp.int32),
            pltpu.VMEM((CHUNK, D), jnp.float32),
            pltpu.VMEM((CHUNK, D), jnp.float32),
            pltpu.VMEM_SHARED((ACC_ROWS, D), jnp.float32),
            pltpu.SemaphoreType.DMA,
            pltpu.SemaphoreType.DMA,
            pltpu.SemaphoreType.DMA,
        ],
        compiler_params=pltpu.CompilerParams(use_tc_tiling_on_sc=False),
    )
    def agg_kernel(x_hbm, ei_hbm, zeros_hbm, out_hbm,
                   src_v, dst_v, rows0, rows1, agg_sh, sem0, sem1, semi):
        c = lax.axis_index("c")
        s = lax.axis_index("s")
        wid = c * NUM_SUBCORES + s

        @pl.when(c == 0)
        def _():
            pltpu.async_copy(x_hbm.at[pl.ds(s * R_PER_T, R_PER_T)],
                             agg_sh.at[pl.ds(s * R_PER_T, R_PER_T)], semi)

            @pl.when(s == NUM_SUBCORES - 1)
            def _():
                pltpu.async_copy(x_hbm.at[pl.ds(TAIL_OFF, R_TAIL)],
                                 agg_sh.at[pl.ds(TAIL_OFF, R_TAIL)], semi)

        @pl.when(c != 0)
        def _():
            pltpu.async_copy(zeros_hbm,
                             agg_sh.at[pl.ds(s * R_PER_T, R_PER_T)], semi)

            @pl.when(s == NUM_SUBCORES - 1)
            def _():
                pltpu.async_copy(zeros_hbm.at[pl.ds(0, R_TAIL)],
                                 agg_sh.at[pl.ds(TAIL_OFF, R_TAIL)], semi)

        pltpu.sync_copy(ei_hbm.at[0, pl.ds(wid * E_PER_W, E_PER_W)], src_v)
        pltpu.sync_copy(ei_hbm.at[1, pl.ds(wid * E_PER_W, E_PER_W)], dst_v)
        pltpu.async_copy(x_hbm.at[src_v.at[pl.ds(0, CHUNK)]], rows0, sem0)

        @pl.when(c == 0)
        def _():
            pltpu.make_async_copy(
                x_hbm.at[pl.ds(s * R_PER_T, R_PER_T)],
                agg_sh.at[pl.ds(s * R_PER_T, R_PER_T)], semi).wait()

            @pl.when(s == NUM_SUBCORES - 1)
            def _():
                pltpu.make_async_copy(
                    x_hbm.at[pl.ds(TAIL_OFF, R_TAIL)],
                    agg_sh.at[pl.ds(TAIL_OFF, R_TAIL)], semi).wait()

        @pl.when(c != 0)
        def _():
            pltpu.make_async_copy(
                zeros_hbm, agg_sh.at[pl.ds(s * R_PER_T, R_PER_T)], semi).wait()

            @pl.when(s == NUM_SUBCORES - 1)
            def _():
                pltpu.make_async_copy(
                    zeros_hbm.at[pl.ds(0, R_TAIL)],
                    agg_sh.at[pl.ds(TAIL_OFF, R_TAIL)], semi).wait()

        plsc.subcore_barrier()

        def src_at(i):
            return src_v.at[pl.ds(i * CHUNK, CHUNK)]

        def dst_at(i):
            return dst_v.at[pl.ds(i * CHUNK, CHUNK)]

        def pair_body(j, carry):
            i = 2 * j
            pltpu.async_copy(x_hbm.at[src_at(i + 1)], rows1, sem1)
            pltpu.make_async_copy(x_hbm.at[src_at(i)], rows0, sem0).wait()
            pltpu.sync_copy(rows0, agg_sh.at[dst_at(i)], add=True)
            pltpu.async_copy(x_hbm.at[src_at(i + 2)], rows0, sem0)
            pltpu.make_async_copy(x_hbm.at[src_at(i + 1)], rows1, sem1).wait()
            pltpu.sync_copy(rows1, agg_sh.at[dst_at(i + 1)], add=True)
            return carry

        lax.fori_loop(0, (NCHUNK - 1) // 2, pair_body, 0)

        last = NCHUNK - 1
        pltpu.make_async_copy(x_hbm.at[src_at(last)], rows0, sem0).wait()
        pltpu.sync_copy(rows0, agg_sh.at[dst_at(last)], add=True)

        plsc.subcore_barrier()

        pltpu.sync_copy(agg_sh.at[pl.ds(s * R_PER_T, R_PER_T)],
                        out_hbm.at[c, pl.ds(s * R_PER_T, R_PER_T)])

        @pl.when(s == NUM_SUBCORES - 1)
        def _():
            pltpu.sync_copy(agg_sh.at[pl.ds(TAIL_OFF, R_TAIL)],
                            out_hbm.at[c, pl.ds(TAIL_OFF, R_TAIL)])

    return agg_kernel(x, ei, zeros)


def _tc_mlp(parts, W1, b1, W2, b2):
    BR = 1000
    grid = N_NODES // BR

    def mlp_body(p_ref, w1_ref, b1_ref, w2_ref, b2_ref, o_ref):
        h = p_ref[0] + p_ref[1]
        h = jnp.dot(h, w1_ref[...], preferred_element_type=jnp.float32)
        h = jnp.maximum(h + b1_ref[...], 0.0)
        h = jnp.dot(h, w2_ref[...], preferred_element_type=jnp.float32)
        o_ref[...] = h + b2_ref[...]

    return pl.pallas_call(
        mlp_body,
        grid=(grid,),
        in_specs=[
            pl.BlockSpec((NUM_CORES, BR, D), lambda i: (0, i, 0)),
            pl.BlockSpec((D, D), lambda i: (0, 0)),
            pl.BlockSpec((1, D), lambda i: (0, 0)),
            pl.BlockSpec((D, D), lambda i: (0, 0)),
            pl.BlockSpec((1, D), lambda i: (0, 0)),
        ],
        out_specs=pl.BlockSpec((BR, D), lambda i: (i, 0)),
        out_shape=jax.ShapeDtypeStruct((N_NODES, D), jnp.float32),
    )(parts, W1, b1, W2, b2)


def kernel(x, edge_index, W1, b1, W2, b2):
    ei = edge_index.astype(jnp.int32)
    zeros = jnp.zeros((R_PER_T, D), jnp.float32)
    parts = _sc_aggregate(x, ei, zeros)
    return _tc_mlp(parts, W1.astype(jnp.float32), b1.reshape(1, D),
                   W2.astype(jnp.float32), b2.reshape(1, D))
